# baseline (speedup 1.0000x reference)
import functools

import jax
import jax.numpy as jnp
import numpy as np
from jax import lax
from jax.experimental import pallas as pl
from jax.experimental.pallas import tpu as pltpu
from jax.experimental.pallas import tpu_sc as plsc

N = 10000
E = 320000
DF = 128
H1 = 16
C = 40
C_PAD = 48

NC = 2
NS = 16
NW = NC * NS
L = 16

CHUNK = 128
G = 80
EPAD = NW * G * CHUNK
NPAD = 10112
RPT = NPAD // NS

_mesh = plsc.VectorSubcoreMesh(
    core_axis_name="c", subcore_axis_name="s", num_cores=NC, num_subcores=NS)
_sc_params = pltpu.CompilerParams(use_tc_tiling_on_sc=False)


def _zero_my_slice(bounce, acc, sid, d):
  zero = jnp.zeros((L,), jnp.float32)

  def zrow(i, carry):
    for j in range(d // L):
      bounce[i, pl.ds(j * L, L)] = zero
    return carry

  lax.fori_loop(0, RPT, zrow, 0)
  pltpu.sync_copy(bounce, acc.at[pl.ds(sid * RPT, RPT)])


def _copy_out_my_slice(bounce, acc, out_hbm, cid, sid):
  pltpu.sync_copy(acc.at[pl.ds(sid * RPT, RPT)], bounce)
  pltpu.sync_copy(bounce, out_hbm.at[cid].at[pl.ds(sid * RPT, RPT)])


@functools.partial(
    pl.kernel,
    out_type=jax.ShapeDtypeStruct((NC, NPAD, L), jnp.float32),
    mesh=_mesh,
    scratch_types=[
        pltpu.VMEM((G, CHUNK), jnp.int32),
        pltpu.VMEM((CHUNK, L), jnp.float32),
        pltpu.VMEM((RPT, L), jnp.float32),
        pltpu.VMEM_SHARED((NPAD, L), jnp.float32),
        pltpu.SemaphoreType.DMA,
        pltpu.SemaphoreType.DMA,
        pltpu.SemaphoreType.DMA,
        pltpu.SemaphoreType.DMA,
    ],
    compiler_params=_sc_params,
)
def _deg_sc(ed_hbm, out_hbm, didx, ones_v, bounce, acc, s0, s1, s2, s3):
  cid = lax.axis_index("c")
  sid = lax.axis_index("s")
  wid = cid * NS + sid
  pltpu.sync_copy(ed_hbm.at[1].at[wid], didx)

  one = jnp.full((L,), 1.0, jnp.float32)

  def orow(i, carry):
    ones_v[i, :] = one
    return carry

  lax.fori_loop(0, CHUNK, orow, 0)
  _zero_my_slice(bounce, acc, sid, L)
  plsc.subcore_barrier()

  sems = (s0, s1, s2, s3)
  for b in range(4):
    pltpu.async_copy(ones_v, acc.at[didx.at[b]], sems[b], add=True)

  def body(j, carry):
    for b in range(4):
      c = 4 * j + 4 + b
      pltpu.make_async_copy(ones_v, acc.at[didx.at[b]], sems[b]).wait()
      pltpu.async_copy(ones_v, acc.at[didx.at[c]], sems[b], add=True)
    return carry

  lax.fori_loop(0, G // 4 - 1, body, 0)
  for b in range(4):
    pltpu.make_async_copy(ones_v, acc.at[didx.at[b]], sems[b]).wait()
  plsc.subcore_barrier()
  _copy_out_my_slice(bounce, acc, out_hbm, cid, sid)


def _make_agg(d):
  @functools.partial(
      pl.kernel,
      out_type=jax.ShapeDtypeStruct((NC, NPAD, d), jnp.float32),
      mesh=_mesh,
      scratch_types=[
          pltpu.VMEM((G, CHUNK), jnp.int32),
          pltpu.VMEM((G, CHUNK), jnp.int32),
          pltpu.VMEM((CHUNK, d), jnp.float32),
          pltpu.VMEM((CHUNK, d), jnp.float32),
          pltpu.VMEM((RPT, d), jnp.float32),
          pltpu.VMEM_SHARED((NPAD, d), jnp.float32),
          pltpu.SemaphoreType.DMA,
          pltpu.SemaphoreType.DMA,
          pltpu.SemaphoreType.DMA,
          pltpu.SemaphoreType.DMA,
      ],
      compiler_params=_sc_params,
  )
  def agg(hs_hbm, ed_hbm, out_hbm, sidx, didx, r0, r1, bounce, acc,
          gs0, gs1, ss0, ss1):
    cid = lax.axis_index("c")
    sid = lax.axis_index("s")
    wid = cid * NS + sid
    pltpu.sync_copy(ed_hbm.at[0].at[wid], sidx)
    pltpu.sync_copy(ed_hbm.at[1].at[wid], didx)
    _zero_my_slice(bounce, acc, sid, d)
    plsc.subcore_barrier()

    pltpu.async_copy(hs_hbm.at[sidx.at[0]], r0, gs0)
    pltpu.async_copy(hs_hbm.at[sidx.at[1]], r1, gs1)

    def body(kk, carry):
      g0 = 2 * kk
      g1 = g0 + 1
      pltpu.make_async_copy(hs_hbm.at[sidx.at[g0]], r0, gs0).wait()
      pltpu.async_copy(r0, acc.at[didx.at[g0]], ss0, add=True)
      pltpu.make_async_copy(hs_hbm.at[sidx.at[g1]], r1, gs1).wait()
      pltpu.async_copy(r1, acc.at[didx.at[g1]], ss1, add=True)

      @pl.when(g0 + 2 < G)
      def _():
        pltpu.make_async_copy(r0, acc.at[didx.at[g0]], ss0).wait()
        pltpu.async_copy(hs_hbm.at[sidx.at[g0 + 2]], r0, gs0)

      @pl.when(g1 + 2 < G)
      def _():
        pltpu.make_async_copy(r1, acc.at[didx.at[g1]], ss1).wait()
        pltpu.async_copy(hs_hbm.at[sidx.at[g1 + 2]], r1, gs1)

      return carry

    lax.fori_loop(0, G // 2, body, 0)
    pltpu.make_async_copy(r0, acc.at[didx.at[0]], ss0).wait()
    pltpu.make_async_copy(r1, acc.at[didx.at[1]], ss1).wait()
    plsc.subcore_barrier()
    _copy_out_my_slice(bounce, acc, out_hbm, cid, sid)

  return agg


_agg16 = _make_agg(H1)
_agg48 = _make_agg(C_PAD)


PR = NPAD * H1 // 128
BLKP = PR // 2

_BB = np.zeros((128, 8 * C_PAD), dtype=np.float32)
for _s in range(8):
  _BB[16 * _s, C_PAD * _s:C_PAD * (_s + 1)] = 1.0
_BB.setflags(write=False)

_FILL = np.asarray(N + np.arange(EPAD - E) % (NPAD - N), dtype=np.int32)
_FILL.setflags(write=False)


def _dinv_of(dp_ref):
  return lax.rsqrt(dp_ref[0] + dp_ref[1] + 1.0)


def _tc1_body(x_ref, w1b_ref, dp_ref, hs1_ref):
  h = jnp.dot(x_ref[...], w1b_ref[...], preferred_element_type=jnp.float32)
  hs1_ref[...] = _dinv_of(dp_ref) * h


def _tc1(xr, w1b, dpP):
  return pl.pallas_call(
      _tc1_body,
      grid=(2,),
      in_specs=[
          pl.BlockSpec((BLKP, 8 * DF), lambda i: (i, 0)),
          pl.BlockSpec((8 * DF, 128), lambda i: (0, 0)),
          pl.BlockSpec((NC, BLKP, 128), lambda i: (0, i, 0)),
      ],
      out_specs=pl.BlockSpec((BLKP, 128), lambda i: (i, 0)),
      out_shape=jax.ShapeDtypeStruct((PR, 128), jnp.float32),
  )(xr, w1b, dpP)


def _tc2_body(a_ref, hs1_ref, dp_ref, w2b_ref, b1t_ref, bb_ref, hs2_ref):
  dinv = _dinv_of(dp_ref)
  s = a_ref[0] + a_ref[1] + hs1_ref[...]
  out1 = jnp.maximum(dinv * s + b1t_ref[...], 0.0)
  h2 = jnp.dot(out1, w2b_ref[...], preferred_element_type=jnp.float32)
  dinv48 = jnp.dot(dinv, bb_ref[...], preferred_element_type=jnp.float32,
                   precision=lax.Precision.HIGHEST)
  hs2_ref[...] = dinv48 * h2


def _tc2(a1P, hs1P, dpP, w2b, b1t, bb):
  return pl.pallas_call(
      _tc2_body,
      grid=(2,),
      in_specs=[
          pl.BlockSpec((NC, BLKP, 128), lambda i: (0, i, 0)),
          pl.BlockSpec((BLKP, 128), lambda i: (i, 0)),
          pl.BlockSpec((NC, BLKP, 128), lambda i: (0, i, 0)),
          pl.BlockSpec((128, 8 * C_PAD), lambda i: (0, 0)),
          pl.BlockSpec((1, 128), lambda i: (0, 0)),
          pl.BlockSpec((128, 8 * C_PAD), lambda i: (0, 0)),
      ],
      out_specs=pl.BlockSpec((BLKP, 8 * C_PAD), lambda i: (i, 0)),
      out_shape=jax.ShapeDtypeStruct((PR, 8 * C_PAD), jnp.float32),
  )(a1P, hs1P, dpP, w2b, b1t, bb)


def _tc3_body(a_ref, hs2_ref, dp_ref, bb_ref, b2t_ref, out_ref):
  dinv = _dinv_of(dp_ref)
  dinv48 = jnp.dot(dinv, bb_ref[...], preferred_element_type=jnp.float32,
                   precision=lax.Precision.HIGHEST)
  s = a_ref[0] + a_ref[1] + hs2_ref[...]
  out_ref[...] = dinv48 * s + b2t_ref[...]


def _tc3(a2P, hs2P, dpP, bb, b2t):
  return pl.pallas_call(
      _tc3_body,
      grid=(2,),
      in_specs=[
          pl.BlockSpec((NC, BLKP, 8 * C_PAD), lambda i: (0, i, 0)),
          pl.BlockSpec((BLKP, 8 * C_PAD), lambda i: (i, 0)),
          pl.BlockSpec((NC, BLKP, 128), lambda i: (0, i, 0)),
          pl.BlockSpec((128, 8 * C_PAD), lambda i: (0, 0)),
          pl.BlockSpec((1, 8 * C_PAD), lambda i: (0, 0)),
      ],
      out_specs=pl.BlockSpec((BLKP, 8 * C_PAD), lambda i: (i, 0)),
      out_shape=jax.ShapeDtypeStruct((PR, 8 * C_PAD), jnp.float32),
  )(a2P, hs2P, dpP, bb, b2t)


def kernel(x, edge_index, W1, b1, W2, b2):
  f32 = jnp.float32
  fill2 = jnp.broadcast_to(jnp.asarray(_FILL), (2, EPAD - E))
  ed = jnp.concatenate([edge_index, fill2], axis=1).reshape(2, NW, G, CHUNK)

  eye8 = jnp.eye(8, dtype=f32)
  xr = jnp.pad(x, ((0, NPAD - N), (0, 0))).reshape(PR, 8 * DF)
  w1b = jnp.kron(eye8, W1)
  w2b = jnp.kron(eye8, jnp.pad(W2, ((0, 0), (0, C_PAD - C))))
  b1t = jnp.tile(b1, 8).reshape(1, 128)
  b2t = jnp.tile(jnp.pad(b2, (0, C_PAD - C)), 8).reshape(1, 8 * C_PAD)
  bb = jnp.asarray(_BB)

  dp = _deg_sc(ed)
  dpP = dp.reshape(NC, PR, 128)
  hs1P = _tc1(xr, w1b, dpP)
  a1 = _agg16(hs1P.reshape(NPAD, H1), ed)
  a1P = a1.reshape(NC, PR, 128)
  hs2P = _tc2(a1P, hs1P, dpP, w2b, b1t, bb)
  a2 = _agg48(hs2P.reshape(NPAD, C_PAD), ed)
  a2P = a2.reshape(NC, PR, 8 * C_PAD)
  oP = _tc3(a2P, hs2P, dpP, bb, b2t)
  return oP.reshape(NPAD, C_PAD)[:N, :C]


# speedup vs baseline: 1.2311x; 1.2311x over previous
import functools

import jax
import jax.numpy as jnp
import numpy as np
from jax import lax
from jax.experimental import pallas as pl
from jax.experimental.pallas import tpu as pltpu
from jax.experimental.pallas import tpu_sc as plsc

N = 10000
E = 320000
DF = 128
H1 = 16
C = 40
C_PAD = 48

NC = 2
NS = 16
NW = NC * NS
L = 16

CHUNK = 128
G = 80
EPAD = NW * G * CHUNK
NPAD = 10112
RPT = NPAD // NS

_mesh = plsc.VectorSubcoreMesh(
    core_axis_name="c", subcore_axis_name="s", num_cores=NC, num_subcores=NS)
_sc_params = pltpu.CompilerParams(use_tc_tiling_on_sc=False)


def _zero_my_slice(bounce, acc, sid, d):
  zero = jnp.zeros((L,), jnp.float32)

  def zrow(i, carry):
    for j in range(d // L):
      bounce[i, pl.ds(j * L, L)] = zero
    return carry

  lax.fori_loop(0, RPT, zrow, 0)
  pltpu.sync_copy(bounce, acc.at[pl.ds(sid * RPT, RPT)])


def _copy_out_my_slice(bounce, acc, out_hbm, cid, sid):
  pltpu.sync_copy(acc.at[pl.ds(sid * RPT, RPT)], bounce)
  pltpu.sync_copy(bounce, out_hbm.at[cid].at[pl.ds(sid * RPT, RPT)])


@functools.partial(
    pl.kernel,
    out_type=jax.ShapeDtypeStruct((NC, NPAD, L), jnp.float32),
    mesh=_mesh,
    scratch_types=[
        pltpu.VMEM((G, CHUNK), jnp.int32),
        pltpu.VMEM((CHUNK, L), jnp.float32),
        pltpu.VMEM((RPT, L), jnp.float32),
        pltpu.VMEM_SHARED((NPAD, L), jnp.float32),
        pltpu.SemaphoreType.DMA,
        pltpu.SemaphoreType.DMA,
        pltpu.SemaphoreType.DMA,
        pltpu.SemaphoreType.DMA,
    ],
    compiler_params=_sc_params,
)
def _deg_sc(ed_hbm, out_hbm, didx, ones_v, bounce, acc, s0, s1, s2, s3):
  cid = lax.axis_index("c")
  sid = lax.axis_index("s")
  wid = cid * NS + sid
  pltpu.sync_copy(ed_hbm.at[1].at[wid], didx)

  one = jnp.full((L,), 1.0, jnp.float32)

  def orow(i, carry):
    ones_v[i, :] = one
    return carry

  lax.fori_loop(0, CHUNK, orow, 0)
  _zero_my_slice(bounce, acc, sid, L)
  plsc.subcore_barrier()

  sems = (s0, s1, s2, s3)
  for b in range(4):
    pltpu.async_copy(ones_v, acc.at[didx.at[b]], sems[b], add=True)

  def body(j, carry):
    for b in range(4):
      c = 4 * j + 4 + b
      pltpu.make_async_copy(ones_v, acc.at[didx.at[b]], sems[b]).wait()
      pltpu.async_copy(ones_v, acc.at[didx.at[c]], sems[b], add=True)
    return carry

  lax.fori_loop(0, G // 4 - 1, body, 0)
  for b in range(4):
    pltpu.make_async_copy(ones_v, acc.at[didx.at[b]], sems[b]).wait()
  plsc.subcore_barrier()
  _copy_out_my_slice(bounce, acc, out_hbm, cid, sid)


def _make_agg(d):
  @functools.partial(
      pl.kernel,
      out_type=jax.ShapeDtypeStruct((NC, NPAD, d), jnp.float32),
      mesh=_mesh,
      scratch_types=[
          pltpu.VMEM((G, CHUNK), jnp.int32),
          pltpu.VMEM((G, CHUNK), jnp.int32),
          pltpu.VMEM((CHUNK, d), jnp.float32),
          pltpu.VMEM((CHUNK, d), jnp.float32),
          pltpu.VMEM((RPT, d), jnp.float32),
          pltpu.VMEM_SHARED((NPAD, d), jnp.float32),
          pltpu.SemaphoreType.DMA,
          pltpu.SemaphoreType.DMA,
          pltpu.SemaphoreType.DMA,
          pltpu.SemaphoreType.DMA,
      ],
      compiler_params=_sc_params,
  )
  def agg(hs_hbm, ed_hbm, out_hbm, sidx, didx, r0, r1, bounce, acc,
          gs0, gs1, ss0, ss1):
    cid = lax.axis_index("c")
    sid = lax.axis_index("s")
    wid = cid * NS + sid
    pltpu.sync_copy(ed_hbm.at[0].at[wid], sidx)
    pltpu.sync_copy(ed_hbm.at[1].at[wid], didx)
    _zero_my_slice(bounce, acc, sid, d)
    plsc.subcore_barrier()

    pltpu.async_copy(hs_hbm.at[sidx.at[0]], r0, gs0)
    pltpu.async_copy(hs_hbm.at[sidx.at[1]], r1, gs1)

    def body(kk, carry):
      g0 = 2 * kk
      g1 = g0 + 1
      pltpu.make_async_copy(hs_hbm.at[sidx.at[g0]], r0, gs0).wait()
      pltpu.async_copy(r0, acc.at[didx.at[g0]], ss0, add=True)
      pltpu.make_async_copy(hs_hbm.at[sidx.at[g1]], r1, gs1).wait()
      pltpu.async_copy(r1, acc.at[didx.at[g1]], ss1, add=True)

      @pl.when(g0 + 2 < G)
      def _():
        pltpu.make_async_copy(r0, acc.at[didx.at[g0]], ss0).wait()
        pltpu.async_copy(hs_hbm.at[sidx.at[g0 + 2]], r0, gs0)

      @pl.when(g1 + 2 < G)
      def _():
        pltpu.make_async_copy(r1, acc.at[didx.at[g1]], ss1).wait()
        pltpu.async_copy(hs_hbm.at[sidx.at[g1 + 2]], r1, gs1)

      return carry

    lax.fori_loop(0, G // 2, body, 0)
    pltpu.make_async_copy(r0, acc.at[didx.at[0]], ss0).wait()
    pltpu.make_async_copy(r1, acc.at[didx.at[1]], ss1).wait()
    plsc.subcore_barrier()
    _copy_out_my_slice(bounce, acc, out_hbm, cid, sid)

  return agg


_agg16 = _make_agg(H1)


PR = NPAD * H1 // 128
BLKP = PR // 2

_FILL = np.asarray(N + np.arange(EPAD - E) % (NPAD - N), dtype=np.int32)
_FILL.setflags(write=False)


def _dinv_of(dp_ref):
  return lax.rsqrt(dp_ref[0] + dp_ref[1] + 1.0)


def _tc1_body(x_ref, w1b_ref, dp_ref, hs1_ref):
  h = jnp.dot(x_ref[...], w1b_ref[...], preferred_element_type=jnp.float32)
  hs1_ref[...] = _dinv_of(dp_ref) * h


def _tc1(xr, w1b, dpP):
  return pl.pallas_call(
      _tc1_body,
      grid=(2,),
      in_specs=[
          pl.BlockSpec((BLKP, 8 * DF), lambda i: (i, 0)),
          pl.BlockSpec((8 * DF, 128), lambda i: (0, 0)),
          pl.BlockSpec((NC, BLKP, 128), lambda i: (0, i, 0)),
      ],
      out_specs=pl.BlockSpec((BLKP, 128), lambda i: (i, 0)),
      out_shape=jax.ShapeDtypeStruct((PR, 128), jnp.float32),
  )(xr, w1b, dpP)


def _tc2_body(a_ref, hs1_ref, dp_ref, b1t_ref, gs2_ref):
  dinv = _dinv_of(dp_ref)
  s = a_ref[0] + a_ref[1] + hs1_ref[...]
  out1 = jnp.maximum(dinv * s + b1t_ref[...], 0.0)
  gs2_ref[...] = dinv * out1


def _tc2(a1P, hs1P, dpP, b1t):
  return pl.pallas_call(
      _tc2_body,
      grid=(2,),
      in_specs=[
          pl.BlockSpec((NC, BLKP, 128), lambda i: (0, i, 0)),
          pl.BlockSpec((BLKP, 128), lambda i: (i, 0)),
          pl.BlockSpec((NC, BLKP, 128), lambda i: (0, i, 0)),
          pl.BlockSpec((1, 128), lambda i: (0, 0)),
      ],
      out_specs=pl.BlockSpec((BLKP, 128), lambda i: (i, 0)),
      out_shape=jax.ShapeDtypeStruct((PR, 128), jnp.float32),
  )(a1P, hs1P, dpP, b1t)


def _tc3_body(a_ref, gs2_ref, dp_ref, w2b_ref, b2t_ref, out_ref):
  dinv = _dinv_of(dp_ref)
  s = dinv * (a_ref[0] + a_ref[1] + gs2_ref[...])
  h2 = jnp.dot(s, w2b_ref[...], preferred_element_type=jnp.float32)
  out_ref[...] = h2 + b2t_ref[...]


def _tc3(a2P, gs2P, dpP, w2b, b2t):
  return pl.pallas_call(
      _tc3_body,
      grid=(2,),
      in_specs=[
          pl.BlockSpec((NC, BLKP, 128), lambda i: (0, i, 0)),
          pl.BlockSpec((BLKP, 128), lambda i: (i, 0)),
          pl.BlockSpec((NC, BLKP, 128), lambda i: (0, i, 0)),
          pl.BlockSpec((128, 8 * C_PAD), lambda i: (0, 0)),
          pl.BlockSpec((1, 8 * C_PAD), lambda i: (0, 0)),
      ],
      out_specs=pl.BlockSpec((BLKP, 8 * C_PAD), lambda i: (i, 0)),
      out_shape=jax.ShapeDtypeStruct((PR, 8 * C_PAD), jnp.float32),
  )(a2P, gs2P, dpP, w2b, b2t)


def kernel(x, edge_index, W1, b1, W2, b2):
  f32 = jnp.float32
  fill2 = jnp.broadcast_to(jnp.asarray(_FILL), (2, EPAD - E))
  ed = jnp.concatenate([edge_index, fill2], axis=1).reshape(2, NW, G, CHUNK)

  eye8 = jnp.eye(8, dtype=f32)
  xr = jnp.pad(x, ((0, NPAD - N), (0, 0))).reshape(PR, 8 * DF)
  w1b = jnp.kron(eye8, W1)
  w2b = jnp.kron(eye8, jnp.pad(W2, ((0, 0), (0, C_PAD - C))))
  b1t = jnp.tile(b1, 8).reshape(1, 128)
  b2t = jnp.tile(jnp.pad(b2, (0, C_PAD - C)), 8).reshape(1, 8 * C_PAD)

  dp = _deg_sc(ed)
  dpP = dp.reshape(NC, PR, 128)
  hs1P = _tc1(xr, w1b, dpP)
  a1 = _agg16(hs1P.reshape(NPAD, H1), ed)
  a1P = a1.reshape(NC, PR, 128)
  gs2P = _tc2(a1P, hs1P, dpP, b1t)
  a2 = _agg16(gs2P.reshape(NPAD, H1), ed)
  a2P = a2.reshape(NC, PR, 128)
  oP = _tc3(a2P, gs2P, dpP, w2b, b2t)
  return oP.reshape(NPAD, C_PAD)[:N, :C]


# speedup vs baseline: 1.5553x; 1.2633x over previous
import functools

import jax
import jax.numpy as jnp
import numpy as np
from jax import lax
from jax.experimental import pallas as pl
from jax.experimental.pallas import tpu as pltpu
from jax.experimental.pallas import tpu_sc as plsc

N = 10000
E = 320000
DF = 128
H1 = 16
C = 40
C_PAD = 48

NC = 2
NS = 16
NW = NC * NS
L = 16

CHUNK = 128
G = 80
EPAD = NW * G * CHUNK
NPAD = 10112
RPT = NPAD // NS

_mesh = plsc.VectorSubcoreMesh(
    core_axis_name="c", subcore_axis_name="s", num_cores=NC, num_subcores=NS)
_sc_params = pltpu.CompilerParams(use_tc_tiling_on_sc=False)


def _zero_my_slice(bounce, acc, sid, d):
  zero = jnp.zeros((L,), jnp.float32)

  def zrow(i, carry):
    for j in range(d // L):
      bounce[i, pl.ds(j * L, L)] = zero
    return carry

  lax.fori_loop(0, RPT, zrow, 0)
  pltpu.sync_copy(bounce, acc.at[pl.ds(sid * RPT, RPT)])


def _copy_out_my_slice(bounce, acc, out_hbm, cid, sid):
  pltpu.sync_copy(acc.at[pl.ds(sid * RPT, RPT)], bounce)
  pltpu.sync_copy(bounce, out_hbm.at[cid].at[pl.ds(sid * RPT, RPT)])


@functools.partial(
    pl.kernel,
    out_type=jax.ShapeDtypeStruct((NC, NPAD, L), jnp.float32),
    mesh=_mesh,
    scratch_types=[
        pltpu.VMEM((G, CHUNK), jnp.int32),
        pltpu.VMEM((CHUNK, L), jnp.float32),
        pltpu.VMEM((RPT, L), jnp.float32),
        pltpu.VMEM_SHARED((NPAD, L), jnp.float32),
        pltpu.SemaphoreType.DMA,
        pltpu.SemaphoreType.DMA,
        pltpu.SemaphoreType.DMA,
        pltpu.SemaphoreType.DMA,
    ],
    compiler_params=_sc_params,
)
def _deg_sc(ed_hbm, out_hbm, didx, ones_v, bounce, acc, s0, s1, s2, s3):
  cid = lax.axis_index("c")
  sid = lax.axis_index("s")
  wid = cid * NS + sid
  pltpu.sync_copy(ed_hbm.at[1].at[wid], didx)

  one = jnp.full((L,), 1.0, jnp.float32)

  def orow(i, carry):
    ones_v[i, :] = one
    return carry

  lax.fori_loop(0, CHUNK, orow, 0)
  _zero_my_slice(bounce, acc, sid, L)
  plsc.subcore_barrier()

  sems = (s0, s1, s2, s3)
  for b in range(4):
    pltpu.async_copy(ones_v, acc.at[didx.at[b]], sems[b], add=True)

  def body(j, carry):
    for b in range(4):
      c = 4 * j + 4 + b
      pltpu.make_async_copy(ones_v, acc.at[didx.at[b]], sems[b]).wait()
      pltpu.async_copy(ones_v, acc.at[didx.at[c]], sems[b], add=True)
    return carry

  lax.fori_loop(0, G // 4 - 1, body, 0)
  for b in range(4):
    pltpu.make_async_copy(ones_v, acc.at[didx.at[b]], sems[b]).wait()
  plsc.subcore_barrier()
  _copy_out_my_slice(bounce, acc, out_hbm, cid, sid)


def _make_agg(d):
  @functools.partial(
      pl.kernel,
      out_type=jax.ShapeDtypeStruct((NC, NPAD, d), jnp.float32),
      mesh=_mesh,
      scratch_types=[
          pltpu.VMEM((G, CHUNK), jnp.int32),
          pltpu.VMEM((G, CHUNK), jnp.int32),
          pltpu.VMEM((CHUNK, d), jnp.float32),
          pltpu.VMEM((CHUNK, d), jnp.float32),
          pltpu.VMEM((RPT, d), jnp.float32),
          pltpu.VMEM_SHARED((NPAD, d), jnp.float32),
          pltpu.VMEM_SHARED((NPAD, d), jnp.float32),
          pltpu.SemaphoreType.DMA,
          pltpu.SemaphoreType.DMA,
          pltpu.SemaphoreType.DMA,
          pltpu.SemaphoreType.DMA,
      ],
      compiler_params=_sc_params,
  )
  def agg(hs_hbm, ed_hbm, out_hbm, sidx, didx, r0, r1, bounce, acc, hsp,
          gs0, gs1, ss0, ss1):
    cid = lax.axis_index("c")
    sid = lax.axis_index("s")
    wid = cid * NS + sid
    pltpu.sync_copy(ed_hbm.at[0].at[wid], sidx)
    pltpu.sync_copy(ed_hbm.at[1].at[wid], didx)
    pltpu.sync_copy(hs_hbm.at[pl.ds(sid * RPT, RPT)], bounce)
    pltpu.sync_copy(bounce, hsp.at[pl.ds(sid * RPT, RPT)])
    _zero_my_slice(bounce, acc, sid, d)
    plsc.subcore_barrier()

    pltpu.async_copy(hsp.at[sidx.at[0]], r0, gs0)
    pltpu.async_copy(hsp.at[sidx.at[1]], r1, gs1)

    def body(kk, carry):
      g0 = 2 * kk
      g1 = g0 + 1
      pltpu.make_async_copy(hsp.at[sidx.at[g0]], r0, gs0).wait()
      pltpu.async_copy(r0, acc.at[didx.at[g0]], ss0, add=True)
      pltpu.make_async_copy(hsp.at[sidx.at[g1]], r1, gs1).wait()
      pltpu.async_copy(r1, acc.at[didx.at[g1]], ss1, add=True)

      @pl.when(g0 + 2 < G)
      def _():
        pltpu.make_async_copy(r0, acc.at[didx.at[g0]], ss0).wait()
        pltpu.async_copy(hsp.at[sidx.at[g0 + 2]], r0, gs0)

      @pl.when(g1 + 2 < G)
      def _():
        pltpu.make_async_copy(r1, acc.at[didx.at[g1]], ss1).wait()
        pltpu.async_copy(hsp.at[sidx.at[g1 + 2]], r1, gs1)

      return carry

    lax.fori_loop(0, G // 2, body, 0)
    pltpu.make_async_copy(r0, acc.at[didx.at[0]], ss0).wait()
    pltpu.make_async_copy(r1, acc.at[didx.at[1]], ss1).wait()
    plsc.subcore_barrier()
    _copy_out_my_slice(bounce, acc, out_hbm, cid, sid)

  return agg


_agg16 = _make_agg(H1)


PR = NPAD * H1 // 128
BLKP = PR // 2

_FILL = np.asarray(N + np.arange(EPAD - E) % (NPAD - N), dtype=np.int32)
_FILL.setflags(write=False)


def _dinv_of(dp_ref):
  return lax.rsqrt(dp_ref[0] + dp_ref[1] + 1.0)


def _tc1_body(x_ref, w1b_ref, dp_ref, hs1_ref):
  h = jnp.dot(x_ref[...], w1b_ref[...], preferred_element_type=jnp.float32)
  hs1_ref[...] = _dinv_of(dp_ref) * h


def _tc1(xr, w1b, dpP):
  return pl.pallas_call(
      _tc1_body,
      grid=(2,),
      in_specs=[
          pl.BlockSpec((BLKP, 8 * DF), lambda i: (i, 0)),
          pl.BlockSpec((8 * DF, 128), lambda i: (0, 0)),
          pl.BlockSpec((NC, BLKP, 128), lambda i: (0, i, 0)),
      ],
      out_specs=pl.BlockSpec((BLKP, 128), lambda i: (i, 0)),
      out_shape=jax.ShapeDtypeStruct((PR, 128), jnp.float32),
  )(xr, w1b, dpP)


def _tc2_body(a_ref, hs1_ref, dp_ref, b1t_ref, gs2_ref):
  dinv = _dinv_of(dp_ref)
  s = a_ref[0] + a_ref[1] + hs1_ref[...]
  out1 = jnp.maximum(dinv * s + b1t_ref[...], 0.0)
  gs2_ref[...] = dinv * out1


def _tc2(a1P, hs1P, dpP, b1t):
  return pl.pallas_call(
      _tc2_body,
      grid=(2,),
      in_specs=[
          pl.BlockSpec((NC, BLKP, 128), lambda i: (0, i, 0)),
          pl.BlockSpec((BLKP, 128), lambda i: (i, 0)),
          pl.BlockSpec((NC, BLKP, 128), lambda i: (0, i, 0)),
          pl.BlockSpec((1, 128), lambda i: (0, 0)),
      ],
      out_specs=pl.BlockSpec((BLKP, 128), lambda i: (i, 0)),
      out_shape=jax.ShapeDtypeStruct((PR, 128), jnp.float32),
  )(a1P, hs1P, dpP, b1t)


def _tc3_body(a_ref, gs2_ref, dp_ref, w2b_ref, b2t_ref, out_ref):
  dinv = _dinv_of(dp_ref)
  s = dinv * (a_ref[0] + a_ref[1] + gs2_ref[...])
  h2 = jnp.dot(s, w2b_ref[...], preferred_element_type=jnp.float32)
  out_ref[...] = h2 + b2t_ref[...]


def _tc3(a2P, gs2P, dpP, w2b, b2t):
  return pl.pallas_call(
      _tc3_body,
      grid=(2,),
      in_specs=[
          pl.BlockSpec((NC, BLKP, 128), lambda i: (0, i, 0)),
          pl.BlockSpec((BLKP, 128), lambda i: (i, 0)),
          pl.BlockSpec((NC, BLKP, 128), lambda i: (0, i, 0)),
          pl.BlockSpec((128, 8 * C_PAD), lambda i: (0, 0)),
          pl.BlockSpec((1, 8 * C_PAD), lambda i: (0, 0)),
      ],
      out_specs=pl.BlockSpec((BLKP, 8 * C_PAD), lambda i: (i, 0)),
      out_shape=jax.ShapeDtypeStruct((PR, 8 * C_PAD), jnp.float32),
  )(a2P, gs2P, dpP, w2b, b2t)


def kernel(x, edge_index, W1, b1, W2, b2):
  f32 = jnp.float32
  fill2 = jnp.broadcast_to(jnp.asarray(_FILL), (2, EPAD - E))
  ed = jnp.concatenate([edge_index, fill2], axis=1).reshape(2, NW, G, CHUNK)

  eye8 = jnp.eye(8, dtype=f32)
  xr = jnp.pad(x, ((0, NPAD - N), (0, 0))).reshape(PR, 8 * DF)
  w1b = jnp.kron(eye8, W1)
  w2b = jnp.kron(eye8, jnp.pad(W2, ((0, 0), (0, C_PAD - C))))
  b1t = jnp.tile(b1, 8).reshape(1, 128)
  b2t = jnp.tile(jnp.pad(b2, (0, C_PAD - C)), 8).reshape(1, 8 * C_PAD)

  dp = _deg_sc(ed)
  dpP = dp.reshape(NC, PR, 128)
  hs1P = _tc1(xr, w1b, dpP)
  a1 = _agg16(hs1P.reshape(NPAD, H1), ed)
  a1P = a1.reshape(NC, PR, 128)
  gs2P = _tc2(a1P, hs1P, dpP, b1t)
  a2 = _agg16(gs2P.reshape(NPAD, H1), ed)
  a2P = a2.reshape(NC, PR, 128)
  oP = _tc3(a2P, gs2P, dpP, w2b, b2t)
  return oP.reshape(NPAD, C_PAD)[:N, :C]


# speedup vs baseline: 1.5757x; 1.0131x over previous
import functools

import jax
import jax.numpy as jnp
import numpy as np
from jax import lax
from jax.experimental import pallas as pl
from jax.experimental.pallas import tpu as pltpu
from jax.experimental.pallas import tpu_sc as plsc

N = 10000
E = 320000
DF = 128
H1 = 16
C = 40
C_PAD = 48

NC = 2
NS = 16
NW = NC * NS
L = 16

CHUNK = 128
G = 80
EPAD = NW * G * CHUNK
NPAD = 10112
RPT = NPAD // NS

_mesh = plsc.VectorSubcoreMesh(
    core_axis_name="c", subcore_axis_name="s", num_cores=NC, num_subcores=NS)
_sc_params = pltpu.CompilerParams(use_tc_tiling_on_sc=False)


def _zero_my_slice(bounce, acc, sid, d):
  zero = jnp.zeros((L,), jnp.float32)

  def zrow(i, carry):
    for j in range(d // L):
      bounce[i, pl.ds(j * L, L)] = zero
    return carry

  lax.fori_loop(0, RPT, zrow, 0)
  pltpu.sync_copy(bounce, acc.at[pl.ds(sid * RPT, RPT)])


def _copy_out_my_slice(bounce, acc, out_hbm, cid, sid):
  pltpu.sync_copy(acc.at[pl.ds(sid * RPT, RPT)], bounce)
  pltpu.sync_copy(bounce, out_hbm.at[cid].at[pl.ds(sid * RPT, RPT)])


@functools.partial(
    pl.kernel,
    out_type=jax.ShapeDtypeStruct((NC, NPAD, L), jnp.float32),
    mesh=_mesh,
    scratch_types=[
        pltpu.VMEM((G, CHUNK), jnp.int32),
        pltpu.VMEM((CHUNK, L), jnp.float32),
        pltpu.VMEM((RPT, L), jnp.float32),
        pltpu.VMEM_SHARED((NPAD, L), jnp.float32),
        pltpu.SemaphoreType.DMA,
        pltpu.SemaphoreType.DMA,
        pltpu.SemaphoreType.DMA,
        pltpu.SemaphoreType.DMA,
    ],
    compiler_params=_sc_params,
)
def _deg_sc(ed_hbm, out_hbm, didx, ones_v, bounce, acc, s0, s1, s2, s3):
  cid = lax.axis_index("c")
  sid = lax.axis_index("s")
  wid = cid * NS + sid
  pltpu.sync_copy(ed_hbm.at[1].at[wid], didx)

  one = jnp.full((L,), 1.0, jnp.float32)

  def orow(i, carry):
    ones_v[i, :] = one
    return carry

  lax.fori_loop(0, CHUNK, orow, 0)
  _zero_my_slice(bounce, acc, sid, L)
  plsc.subcore_barrier()

  sems = (s0, s1, s2, s3)
  for b in range(4):
    pltpu.async_copy(ones_v, acc.at[didx.at[b]], sems[b], add=True)

  def body(j, carry):
    for b in range(4):
      c = 4 * j + 4 + b
      pltpu.make_async_copy(ones_v, acc.at[didx.at[b]], sems[b]).wait()
      pltpu.async_copy(ones_v, acc.at[didx.at[c]], sems[b], add=True)
    return carry

  lax.fori_loop(0, G // 4 - 1, body, 0)
  for b in range(4):
    pltpu.make_async_copy(ones_v, acc.at[didx.at[b]], sems[b]).wait()
  plsc.subcore_barrier()
  _copy_out_my_slice(bounce, acc, out_hbm, cid, sid)


def _make_agg(d):
  @functools.partial(
      pl.kernel,
      out_type=jax.ShapeDtypeStruct((NC, NPAD, d), jnp.float32),
      mesh=_mesh,
      scratch_types=[
          pltpu.VMEM((G, CHUNK), jnp.int32),
          pltpu.VMEM((G, CHUNK), jnp.int32),
          pltpu.VMEM((CHUNK, d), jnp.float32),
          pltpu.VMEM((CHUNK, d), jnp.float32),
          pltpu.VMEM((RPT, d), jnp.float32),
          pltpu.VMEM_SHARED((NPAD, d), jnp.float32),
          pltpu.VMEM_SHARED((NPAD, d), jnp.float32),
          pltpu.SemaphoreType.DMA,
          pltpu.SemaphoreType.DMA,
          pltpu.SemaphoreType.DMA,
          pltpu.SemaphoreType.DMA,
      ],
      compiler_params=_sc_params,
  )
  def agg(hs_hbm, ed_hbm, out_hbm, sidx, didx, r0, r1, bounce, acc, hsp,
          gs0, gs1, ss0, ss1):
    cid = lax.axis_index("c")
    sid = lax.axis_index("s")
    wid = cid * NS + sid
    pltpu.sync_copy(ed_hbm.at[0].at[wid], sidx)
    pltpu.sync_copy(ed_hbm.at[1].at[wid], didx)
    pltpu.sync_copy(hs_hbm.at[pl.ds(sid * RPT, RPT)], bounce)
    pltpu.sync_copy(bounce, hsp.at[pl.ds(sid * RPT, RPT)])
    _zero_my_slice(bounce, acc, sid, d)
    plsc.subcore_barrier()

    pltpu.async_copy(hsp.at[sidx.at[0]], r0, gs0)
    pltpu.async_copy(hsp.at[sidx.at[1]], r1, gs1)

    def body(kk, carry):
      g0 = 2 * kk
      g1 = g0 + 1
      pltpu.make_async_copy(hsp.at[sidx.at[g0]], r0, gs0).wait()
      pltpu.async_copy(r0, acc.at[didx.at[g0]], ss0, add=True)
      pltpu.make_async_copy(hsp.at[sidx.at[g1]], r1, gs1).wait()
      pltpu.async_copy(r1, acc.at[didx.at[g1]], ss1, add=True)

      @pl.when(g0 + 2 < G)
      def _():
        pltpu.make_async_copy(r0, acc.at[didx.at[g0]], ss0).wait()
        pltpu.async_copy(hsp.at[sidx.at[g0 + 2]], r0, gs0)

      @pl.when(g1 + 2 < G)
      def _():
        pltpu.make_async_copy(r1, acc.at[didx.at[g1]], ss1).wait()
        pltpu.async_copy(hsp.at[sidx.at[g1 + 2]], r1, gs1)

      return carry

    lax.fori_loop(0, G // 2, body, 0)
    pltpu.make_async_copy(r0, acc.at[didx.at[0]], ss0).wait()
    pltpu.make_async_copy(r1, acc.at[didx.at[1]], ss1).wait()
    plsc.subcore_barrier()
    _copy_out_my_slice(bounce, acc, out_hbm, cid, sid)

  return agg


_agg16 = _make_agg(H1)


PR = NPAD * H1 // 128
BLKP = PR // 2

_FILL = np.asarray(N + np.arange(EPAD - E) % (NPAD - N), dtype=np.int32)
_FILL.setflags(write=False)


def _dinv_of(dp_ref):
  return lax.rsqrt(dp_ref[0] + dp_ref[1] + 1.0)


def _tc1a_body(x_ref, w1b_ref, xw1_ref):
  xw1_ref[...] = jnp.dot(x_ref[...], w1b_ref[...],
                         preferred_element_type=jnp.float32)


def _tc1a(xr, w1b):
  return pl.pallas_call(
      _tc1a_body,
      grid=(2,),
      in_specs=[
          pl.BlockSpec((BLKP, 8 * DF), lambda i: (i, 0)),
          pl.BlockSpec((8 * DF, 128), lambda i: (0, 0)),
      ],
      out_specs=pl.BlockSpec((BLKP, 128), lambda i: (i, 0)),
      out_shape=jax.ShapeDtypeStruct((PR, 128), jnp.float32),
  )(xr, w1b)


def _tc1b_body(xw1_ref, dp_ref, hs1_ref):
  hs1_ref[...] = _dinv_of(dp_ref) * xw1_ref[...]


def _tc1b(xw1P, dpP):
  return pl.pallas_call(
      _tc1b_body,
      grid=(2,),
      in_specs=[
          pl.BlockSpec((BLKP, 128), lambda i: (i, 0)),
          pl.BlockSpec((NC, BLKP, 128), lambda i: (0, i, 0)),
      ],
      out_specs=pl.BlockSpec((BLKP, 128), lambda i: (i, 0)),
      out_shape=jax.ShapeDtypeStruct((PR, 128), jnp.float32),
  )(xw1P, dpP)


def _tc2_body(a_ref, hs1_ref, dp_ref, b1t_ref, gs2_ref):
  dinv = _dinv_of(dp_ref)
  s = a_ref[0] + a_ref[1] + hs1_ref[...]
  out1 = jnp.maximum(dinv * s + b1t_ref[...], 0.0)
  gs2_ref[...] = dinv * out1


def _tc2(a1P, hs1P, dpP, b1t):
  return pl.pallas_call(
      _tc2_body,
      grid=(2,),
      in_specs=[
          pl.BlockSpec((NC, BLKP, 128), lambda i: (0, i, 0)),
          pl.BlockSpec((BLKP, 128), lambda i: (i, 0)),
          pl.BlockSpec((NC, BLKP, 128), lambda i: (0, i, 0)),
          pl.BlockSpec((1, 128), lambda i: (0, 0)),
      ],
      out_specs=pl.BlockSpec((BLKP, 128), lambda i: (i, 0)),
      out_shape=jax.ShapeDtypeStruct((PR, 128), jnp.float32),
  )(a1P, hs1P, dpP, b1t)


def _tc3_body(a_ref, gs2_ref, dp_ref, w2b_ref, b2t_ref, out_ref):
  dinv = _dinv_of(dp_ref)
  s = dinv * (a_ref[0] + a_ref[1] + gs2_ref[...])
  h2 = jnp.dot(s, w2b_ref[...], preferred_element_type=jnp.float32)
  out_ref[...] = h2 + b2t_ref[...]


def _tc3(a2P, gs2P, dpP, w2b, b2t):
  return pl.pallas_call(
      _tc3_body,
      grid=(2,),
      in_specs=[
          pl.BlockSpec((NC, BLKP, 128), lambda i: (0, i, 0)),
          pl.BlockSpec((BLKP, 128), lambda i: (i, 0)),
          pl.BlockSpec((NC, BLKP, 128), lambda i: (0, i, 0)),
          pl.BlockSpec((128, 8 * C_PAD), lambda i: (0, 0)),
          pl.BlockSpec((1, 8 * C_PAD), lambda i: (0, 0)),
      ],
      out_specs=pl.BlockSpec((BLKP, 8 * C_PAD), lambda i: (i, 0)),
      out_shape=jax.ShapeDtypeStruct((PR, 8 * C_PAD), jnp.float32),
  )(a2P, gs2P, dpP, w2b, b2t)


def kernel(x, edge_index, W1, b1, W2, b2):
  f32 = jnp.float32
  fill2 = jnp.broadcast_to(jnp.asarray(_FILL), (2, EPAD - E))
  ed = jnp.concatenate([edge_index, fill2], axis=1).reshape(2, NW, G, CHUNK)

  eye8 = jnp.eye(8, dtype=f32)
  xr = jnp.pad(x, ((0, NPAD - N), (0, 0))).reshape(PR, 8 * DF)
  w1b = jnp.kron(eye8, W1)
  w2b = jnp.kron(eye8, jnp.pad(W2, ((0, 0), (0, C_PAD - C))))
  b1t = jnp.tile(b1, 8).reshape(1, 128)
  b2t = jnp.tile(jnp.pad(b2, (0, C_PAD - C)), 8).reshape(1, 8 * C_PAD)

  xw1P = _tc1a(xr, w1b)
  dp = _deg_sc(ed)
  dpP = dp.reshape(NC, PR, 128)
  hs1P = _tc1b(xw1P, dpP)
  a1 = _agg16(hs1P.reshape(NPAD, H1), ed)
  a1P = a1.reshape(NC, PR, 128)
  gs2P = _tc2(a1P, hs1P, dpP, b1t)
  a2 = _agg16(gs2P.reshape(NPAD, H1), ed)
  a2P = a2.reshape(NC, PR, 128)
  oP = _tc3(a2P, gs2P, dpP, w2b, b2t)
  return oP.reshape(NPAD, C_PAD)[:N, :C]
